# baseline (speedup 1.0000x reference)
import functools
import jax
import jax.numpy as jnp
from jax import lax
from jax.experimental import pallas as pl
from jax.experimental.pallas import tpu as pltpu
from jax.experimental.pallas import tpu_sc as plsc

NN = 50000
EG = 800000
G = 1024
D = 72
DW = 80
NB = 512
NP = 50176
EP = 800256
HALF = NP // 2
EPB = 48
SUBS = 16
BLOCKS_PER_SUB = EP // (SUBS * EPB)
CO_CHUNK = 49
ROWS_PER_SUB = HALF // SUBS
MSG_OFFS = (0, 16, 32, 48, 56)

_HI = jax.lax.Precision.HIGHEST


def _onehot(ids, width):
    iot = lax.broadcasted_iota(jnp.int32, (ids.shape[0], width), 1)
    return (ids == iot).astype(jnp.float32)


def _node_embed_body(nf_ref, ta_ref, tc_ref, th_ref, wx_ref, bx_ref, o_ref):
    nf = nf_ref[...]
    ai = nf[:, 0:1].astype(jnp.int32)
    ci = nf[:, 1:2].astype(jnp.int32)
    hi = nf[:, 2:3].astype(jnp.int32)
    x = jnp.dot(_onehot(ai, 32), ta_ref[...], precision=_HI,
                preferred_element_type=jnp.float32)
    x += jnp.dot(_onehot(ci, 32), tc_ref[...], precision=_HI,
                 preferred_element_type=jnp.float32)
    x += jnp.dot(_onehot(hi, 32), th_ref[...], precision=_HI,
                 preferred_element_type=jnp.float32)
    x += jnp.dot(nf[:, 3:6], wx_ref[...], precision=_HI,
                 preferred_element_type=jnp.float32)
    o_ref[...] = x + bx_ref[...]


def _node_embed(nf, ta, tc, th, wx, bx):
    grid = NP // NB
    return pl.pallas_call(
        _node_embed_body,
        grid=(grid,),
        in_specs=[
            pl.BlockSpec((NB, 6), lambda i: (i, 0)),
            pl.BlockSpec((32, D), lambda i: (0, 0)),
            pl.BlockSpec((32, D), lambda i: (0, 0)),
            pl.BlockSpec((32, D), lambda i: (0, 0)),
            pl.BlockSpec((3, D), lambda i: (0, 0)),
            pl.BlockSpec((1, D), lambda i: (0, 0)),
        ],
        out_specs=pl.BlockSpec((NB, D), lambda i: (i, 0)),
        out_shape=jax.ShapeDtypeStruct((NP, D), jnp.float32),
    )(nf, ta, tc, th, wx, bx)


def _edge_q_body(ef_ref, tb_ref, ts_ref, we_ref, be_ref, wm_ref, bm_ref, o_ref):
    ef = ef_ref[...]
    bi = ef[:, 0:1].astype(jnp.int32)
    si = ef[:, 1:2].astype(jnp.int32)
    e = jnp.dot(_onehot(bi, 32), tb_ref[...], precision=_HI,
                preferred_element_type=jnp.float32)
    e += jnp.dot(_onehot(si, 32), ts_ref[...], precision=_HI,
                 preferred_element_type=jnp.float32)
    e += jnp.dot(ef[:, 2:4], we_ref[...], precision=_HI,
                 preferred_element_type=jnp.float32)
    e += be_ref[...]
    o_ref[...] = jnp.dot(e, wm_ref[...], precision=_HI,
                         preferred_element_type=jnp.float32) + bm_ref[...]


def _edge_q(ef, tb, ts, we, be, wm_pad, bm_pad):
    grid = EP // NB
    return pl.pallas_call(
        _edge_q_body,
        grid=(grid,),
        in_specs=[
            pl.BlockSpec((NB, 4), lambda i: (i, 0)),
            pl.BlockSpec((32, 36), lambda i: (0, 0)),
            pl.BlockSpec((32, 36), lambda i: (0, 0)),
            pl.BlockSpec((2, 36), lambda i: (0, 0)),
            pl.BlockSpec((1, 36), lambda i: (0, 0)),
            pl.BlockSpec((36, DW), lambda i: (0, 0)),
            pl.BlockSpec((1, DW), lambda i: (0, 0)),
        ],
        out_specs=pl.BlockSpec((NB, DW), lambda i: (i, 0)),
        out_shape=jax.ShapeDtypeStruct((EP, DW), jnp.float32),
    )(ef, tb, ts, we, be, wm_pad, bm_pad)


def _p_body(x_ref, w_ref, o_ref):
    o_ref[...] = jnp.dot(x_ref[...], w_ref[...], precision=_HI,
                         preferred_element_type=jnp.float32)


def _p_mat(x, w_pad):
    grid = NP // NB
    return pl.pallas_call(
        _p_body,
        grid=(grid,),
        in_specs=[
            pl.BlockSpec((NB, D), lambda i: (i, 0)),
            pl.BlockSpec((D, DW), lambda i: (0, 0)),
        ],
        out_specs=pl.BlockSpec((NB, DW), lambda i: (i, 0)),
        out_shape=jax.ShapeDtypeStruct((NP, DW), jnp.float32),
    )(x, w_pad)


def _upd_body(x_ref, a_ref, w1_ref, w2_ref, b_ref, o_ref):
    v = jnp.dot(x_ref[...], w1_ref[...], precision=_HI,
                preferred_element_type=jnp.float32)
    v += jnp.dot(a_ref[...], w2_ref[...], precision=_HI,
                 preferred_element_type=jnp.float32)
    o_ref[...] = jnp.maximum(v + b_ref[...], 0.0)


def _update(x, agg, wu1, wu2, b):
    grid = NP // NB
    return pl.pallas_call(
        _upd_body,
        grid=(grid,),
        in_specs=[
            pl.BlockSpec((NB, D), lambda i: (i, 0)),
            pl.BlockSpec((NB, D), lambda i: (i, 0)),
            pl.BlockSpec((D, D), lambda i: (0, 0)),
            pl.BlockSpec((D, D), lambda i: (0, 0)),
            pl.BlockSpec((1, D), lambda i: (0, 0)),
        ],
        out_specs=pl.BlockSpec((NB, D), lambda i: (i, 0)),
        out_shape=jax.ShapeDtypeStruct((NP, D), jnp.float32),
    )(x, agg, wu1, wu2, b)


def _sc_round(P, Q, snd, rcv):
    mesh = plsc.VectorSubcoreMesh(core_axis_name="c", subcore_axis_name="s")

    @functools.partial(
        pl.kernel,
        mesh=mesh,
        out_type=jax.ShapeDtypeStruct((NP, D), jnp.float32),
        compiler_params=pltpu.CompilerParams(use_tc_tiling_on_sc=False),
        scratch_types=[
            pltpu.VMEM((EPB,), jnp.int32),
            pltpu.VMEM((EPB,), jnp.int32),
            pltpu.VMEM((EPB,), jnp.int32),
            pltpu.VMEM((EPB, DW), jnp.float32),
            pltpu.VMEM((EPB, DW), jnp.float32),
            pltpu.VMEM((EPB, D), jnp.float32),
            pltpu.VMEM((CO_CHUNK, D), jnp.float32),
            pltpu.VMEM_SHARED((HALF + 8, D), jnp.float32),
            pltpu.SemaphoreType.DMA,
        ],
    )
    def k(p_h, q_h, s_h, r_h, out_h, sidx, ridx, lidx, prow, qrow, mrow, cbuf,
          acc, sem):
        cid = lax.axis_index("c")
        sid = lax.axis_index("s")
        coff = cid * HALF

        def zrow(i, c):
            for j in MSG_OFFS:
                cbuf[i, pl.ds(j, 16)] = jnp.zeros((16,), jnp.float32)
            return c
        lax.fori_loop(0, CO_CHUNK, zrow, 0)

        def zchunk(kk, c):
            pltpu.sync_copy(cbuf, acc.at[pl.ds(sid * ROWS_PER_SUB
                                               + kk * CO_CHUNK, CO_CHUNK)])
            return c
        lax.fori_loop(0, ROWS_PER_SUB // CO_CHUNK, zchunk, 0)

        @pl.when(sid == 0)
        def _():
            pltpu.sync_copy(cbuf.at[pl.ds(0, 8)], acc.at[pl.ds(HALF, 8)])

        plsc.subcore_barrier()

        base0 = sid * (BLOCKS_PER_SUB * EPB)

        def body(b, c):
            base = base0 + b * EPB
            pltpu.sync_copy(s_h.at[pl.ds(base, EPB)], sidx)
            pltpu.sync_copy(r_h.at[pl.ds(base, EPB)], ridx)
            pltpu.async_copy(p_h.at[sidx], prow, sem).wait()
            pltpu.sync_copy(q_h.at[pl.ds(base, EPB)], qrow)

            def lix(i, c2):
                r = ridx[pl.ds(i * 16, 16)] - coff
                ok = (r >= 0) & (r < HALF)
                lidx[pl.ds(i * 16, 16)] = jnp.where(ok, r, HALF)
                return c2
            lax.fori_loop(0, EPB // 16, lix, 0)

            def rrow(i, c2):
                for j in MSG_OFFS:
                    v = prow[i, pl.ds(j, 16)] + qrow[i, pl.ds(j, 16)]
                    mrow[i, pl.ds(j, 16)] = jnp.maximum(v, 0.0)
                return c2
            lax.fori_loop(0, EPB, rrow, 0)

            pltpu.sync_copy(mrow, acc.at[lidx], add=True)
            return c
        lax.fori_loop(0, BLOCKS_PER_SUB, body, 0)

        plsc.subcore_barrier()

        def cout(kk, c):
            off = sid * ROWS_PER_SUB + kk * CO_CHUNK
            pltpu.sync_copy(acc.at[pl.ds(off, CO_CHUNK)], cbuf)
            pltpu.sync_copy(cbuf, out_h.at[pl.ds(coff + off, CO_CHUNK)])
            return c
        lax.fori_loop(0, ROWS_PER_SUB // CO_CHUNK, cout, 0)

    return k(P, Q, snd, rcv)


def _pool1_body(x_ref, wp_ref, idx_ref, lg_ref, mx_ref):
    @pl.when(pl.program_id(0) == 0)
    def _():
        mx_ref[...] = jnp.full((1, G), -1e30, jnp.float32)
    x = x_ref[...]
    idx = idx_ref[...].reshape(NB, 1)
    lg = jnp.dot(x, wp_ref[...], precision=_HI,
                 preferred_element_type=jnp.float32)
    lgm = jnp.where(idx < G, lg, 0.0)
    lg_ref[...] = lgm
    oh = _onehot(idx, G)
    cand = jnp.where(oh > 0.0, lgm, -1e30)
    mx_ref[...] = jnp.maximum(mx_ref[...], jnp.max(cand, axis=0)[None, :])


def _pool1(x, wp, idx3):
    grid = NP // NB
    return pl.pallas_call(
        _pool1_body,
        grid=(grid,),
        in_specs=[
            pl.BlockSpec((NB, D), lambda i: (i, 0)),
            pl.BlockSpec((D, 1), lambda i: (0, 0)),
            pl.BlockSpec((1, 1, NB), lambda i: (i, 0, 0)),
        ],
        out_specs=[
            pl.BlockSpec((NB, 1), lambda i: (i, 0)),
            pl.BlockSpec((1, G), lambda i: (0, 0)),
        ],
        out_shape=[
            jax.ShapeDtypeStruct((NP, 1), jnp.float32),
            jax.ShapeDtypeStruct((1, G), jnp.float32),
        ],
    )(x, wp, idx3)


def _pool2_body(x_ref, lg_ref, mx_ref, idx_ref, s_ref, dn_ref):
    @pl.when(pl.program_id(0) == 0)
    def _():
        s_ref[...] = jnp.zeros((G, D), jnp.float32)
        dn_ref[...] = jnp.zeros((G, 1), jnp.float32)
    x = x_ref[...]
    idx = idx_ref[...].reshape(NB, 1)
    oh = _onehot(idx, G)
    mxn = jnp.dot(oh, mx_ref[...].reshape(G, 1), precision=_HI,
                  preferred_element_type=jnp.float32)
    ex = jnp.exp(lg_ref[...] - mxn)
    s_ref[...] += jnp.dot(oh.T, ex * x, precision=_HI,
                          preferred_element_type=jnp.float32)
    dn_ref[...] += jnp.dot(oh.T, ex, precision=_HI,
                           preferred_element_type=jnp.float32)


def _pool2(x, lg, mx, idx3):
    grid = NP // NB
    return pl.pallas_call(
        _pool2_body,
        grid=(grid,),
        in_specs=[
            pl.BlockSpec((NB, D), lambda i: (i, 0)),
            pl.BlockSpec((NB, 1), lambda i: (i, 0)),
            pl.BlockSpec((1, G), lambda i: (0, 0)),
            pl.BlockSpec((1, 1, NB), lambda i: (i, 0, 0)),
        ],
        out_specs=[
            pl.BlockSpec((G, D), lambda i: (0, 0)),
            pl.BlockSpec((G, 1), lambda i: (0, 0)),
        ],
        out_shape=[
            jax.ShapeDtypeStruct((G, D), jnp.float32),
            jax.ShapeDtypeStruct((G, 1), jnp.float32),
        ],
    )(x, lg, mx, idx3)


def _mlp_body(s_ref, dn_ref, w1_ref, b1_ref, w2_ref, b2_ref, w3_ref, b3_ref,
              w4_ref, b4_ref, lg_ref, lb_ref, wo_ref, bo_ref, o_ref):
    g = s_ref[...] / (dn_ref[...] + 1e-9)
    ge = g.reshape(G // 2, 2, D)[:, 0, :]
    x = jnp.maximum(jnp.dot(ge, w1_ref[...], precision=_HI,
                            preferred_element_type=jnp.float32)
                    + b1_ref[...], 0.0)
    x = jnp.maximum(jnp.dot(x, w2_ref[...], precision=_HI,
                            preferred_element_type=jnp.float32)
                    + b2_ref[...], 0.0)
    x = jnp.maximum(jnp.dot(x, w3_ref[...], precision=_HI,
                            preferred_element_type=jnp.float32)
                    + b3_ref[...], 0.0)
    x = jnp.dot(x, w4_ref[...], precision=_HI,
                preferred_element_type=jnp.float32) + b4_ref[...]
    m = jnp.mean(x, axis=-1, keepdims=True)
    v = jnp.mean((x - m) * (x - m), axis=-1, keepdims=True)
    xe = lg_ref[...] * (x - m) * jax.lax.rsqrt(v + 1e-6) + lb_ref[...]
    o_ref[...] = jnp.dot(xe, wo_ref[...], precision=_HI,
                         preferred_element_type=jnp.float32) + bo_ref[...]


def _mlp(s, dn, w1, b1, w2, b2, w3, b3, w4, b4, lng, lnb, wo, bo):
    H = 392
    NCLS = 12
    return pl.pallas_call(
        _mlp_body,
        in_specs=[
            pl.BlockSpec((G, D), lambda: (0, 0)),
            pl.BlockSpec((G, 1), lambda: (0, 0)),
            pl.BlockSpec((D, H), lambda: (0, 0)),
            pl.BlockSpec((1, H), lambda: (0, 0)),
            pl.BlockSpec((H, H), lambda: (0, 0)),
            pl.BlockSpec((1, H), lambda: (0, 0)),
            pl.BlockSpec((H, H), lambda: (0, 0)),
            pl.BlockSpec((1, H), lambda: (0, 0)),
            pl.BlockSpec((H, H), lambda: (0, 0)),
            pl.BlockSpec((1, H), lambda: (0, 0)),
            pl.BlockSpec((1, H), lambda: (0, 0)),
            pl.BlockSpec((1, H), lambda: (0, 0)),
            pl.BlockSpec((H, NCLS), lambda: (0, 0)),
            pl.BlockSpec((1, NCLS), lambda: (0, 0)),
        ],
        out_specs=pl.BlockSpec((G // 2, NCLS), lambda: (0, 0)),
        out_shape=jax.ShapeDtypeStruct((G // 2, NCLS), jnp.float32),
    )(s, dn, w1, b1, w2, b2, w3, b3, w4, b4, lng, lnb, wo, bo)


def kernel(node_feats, edge_feats, senders, receivers, node_graph_idx,
           atomic_num_table, chiral_table, hybrid_table, bond_type_table,
           stereo_table, W_xproj, b_xproj, W_eproj, b_eproj, W_msg, b_msg,
           W_upd, b_upd, W_pool, W1, b1, W2, b2, W3, b3, W4, b4, ln_g, ln_b,
           W_out, b_out, deterministic, return_embeddings):
    f32 = jnp.float32
    nf = jnp.zeros((NP, 6), f32).at[:NN].set(node_feats.astype(f32))
    ef = jnp.zeros((EP, 4), f32).at[:EG].set(edge_feats.astype(f32))
    snd = jnp.zeros((EP,), jnp.int32).at[:EG].set(senders.astype(jnp.int32))
    rcv = jnp.full((EP,), NP, jnp.int32).at[:EG].set(
        receivers.astype(jnp.int32))
    gidx = jnp.full((NP,), G, jnp.int32).at[:NN].set(
        node_graph_idx.astype(jnp.int32))
    gidx3 = gidx.reshape(NP // NB, 1, NB)

    wm_x = jnp.zeros((D, DW), f32).at[:, :D].set(W_msg[:D].astype(f32))
    wm_e = jnp.zeros((36, DW), f32).at[:, :D].set(W_msg[D:].astype(f32))
    bm = jnp.zeros((1, DW), f32).at[0, :D].set(b_msg.astype(f32))
    wu1 = W_upd[:D].astype(f32)
    wu2 = W_upd[D:].astype(f32)

    x = _node_embed(nf, atomic_num_table.astype(f32),
                    chiral_table.astype(f32), hybrid_table.astype(f32),
                    W_xproj.astype(f32), b_xproj.astype(f32).reshape(1, D))
    q = _edge_q(ef, bond_type_table.astype(f32), stereo_table.astype(f32),
                W_eproj.astype(f32), b_eproj.astype(f32).reshape(1, 36),
                wm_e, bm)

    bu = b_upd.astype(f32).reshape(1, D)
    for _ in range(5):
        p = _p_mat(x, wm_x)
        agg = _sc_round(p, q, snd, rcv)
        x = _update(x, agg, wu1, wu2, bu)

    lg, mx = _pool1(x, W_pool.astype(f32), gidx3)
    s, dn = _pool2(x, lg, mx, gidx3)
    out = _mlp(s, dn, W1.astype(f32), b1.astype(f32).reshape(1, 392),
               W2.astype(f32), b2.astype(f32).reshape(1, 392),
               W3.astype(f32), b3.astype(f32).reshape(1, 392),
               W4.astype(f32), b4.astype(f32).reshape(1, 392),
               ln_g.astype(f32).reshape(1, 392), ln_b.astype(f32).reshape(1, 392),
               W_out.astype(f32), b_out.astype(f32).reshape(1, 12))
    return out + jnp.zeros((), out.dtype) * return_embeddings


# speedup vs baseline: 1.2687x; 1.2687x over previous
import functools
import jax
import jax.numpy as jnp
from jax import lax
from jax.experimental import pallas as pl
from jax.experimental.pallas import tpu as pltpu
from jax.experimental.pallas import tpu_sc as plsc

NN = 50000
EG = 800000
G = 1024
D = 72
DW = 80
NB = 512
NP = 50176
EP = 800768
HALF = NP // 2
EPB = 32
SUBS = 16
PAIRS = EP // (SUBS * EPB * 2)
CO_CHUNK = 49
ROWS_PER_SUB = HALF // SUBS
MSG_OFFS = (0, 16, 32, 48, 56)

_HI = jax.lax.Precision.HIGHEST


def _onehot(ids, width):
    iot = lax.broadcasted_iota(jnp.int32, (ids.shape[0], width), 1)
    return (ids == iot).astype(jnp.float32)


def _node_embed_body(nf_ref, ta_ref, tc_ref, th_ref, wx_ref, bx_ref, o_ref):
    nf = nf_ref[...]
    ai = nf[:, 0:1].astype(jnp.int32)
    ci = nf[:, 1:2].astype(jnp.int32)
    hi = nf[:, 2:3].astype(jnp.int32)
    x = jnp.dot(_onehot(ai, 32), ta_ref[...], precision=_HI,
                preferred_element_type=jnp.float32)
    x += jnp.dot(_onehot(ci, 32), tc_ref[...], precision=_HI,
                 preferred_element_type=jnp.float32)
    x += jnp.dot(_onehot(hi, 32), th_ref[...], precision=_HI,
                 preferred_element_type=jnp.float32)
    x += jnp.dot(nf[:, 3:6], wx_ref[...], precision=_HI,
                 preferred_element_type=jnp.float32)
    o_ref[...] = x + bx_ref[...]


def _node_embed(nf, ta, tc, th, wx, bx):
    grid = NP // NB
    return pl.pallas_call(
        _node_embed_body,
        grid=(grid,),
        in_specs=[
            pl.BlockSpec((NB, 6), lambda i: (i, 0)),
            pl.BlockSpec((32, D), lambda i: (0, 0)),
            pl.BlockSpec((32, D), lambda i: (0, 0)),
            pl.BlockSpec((32, D), lambda i: (0, 0)),
            pl.BlockSpec((3, D), lambda i: (0, 0)),
            pl.BlockSpec((1, D), lambda i: (0, 0)),
        ],
        out_specs=pl.BlockSpec((NB, D), lambda i: (i, 0)),
        out_shape=jax.ShapeDtypeStruct((NP, D), jnp.float32),
    )(nf, ta, tc, th, wx, bx)


def _edge_q_body(ef_ref, tb_ref, ts_ref, we_ref, be_ref, wm_ref, bm_ref, o_ref):
    ef = ef_ref[...]
    bi = ef[:, 0:1].astype(jnp.int32)
    si = ef[:, 1:2].astype(jnp.int32)
    e = jnp.dot(_onehot(bi, 32), tb_ref[...], precision=_HI,
                preferred_element_type=jnp.float32)
    e += jnp.dot(_onehot(si, 32), ts_ref[...], precision=_HI,
                 preferred_element_type=jnp.float32)
    e += jnp.dot(ef[:, 2:4], we_ref[...], precision=_HI,
                 preferred_element_type=jnp.float32)
    e += be_ref[...]
    o_ref[...] = jnp.dot(e, wm_ref[...], precision=_HI,
                         preferred_element_type=jnp.float32) + bm_ref[...]


def _edge_q(ef, tb, ts, we, be, wm_pad, bm_pad):
    grid = EP // NB
    return pl.pallas_call(
        _edge_q_body,
        grid=(grid,),
        in_specs=[
            pl.BlockSpec((NB, 4), lambda i: (i, 0)),
            pl.BlockSpec((32, 36), lambda i: (0, 0)),
            pl.BlockSpec((32, 36), lambda i: (0, 0)),
            pl.BlockSpec((2, 36), lambda i: (0, 0)),
            pl.BlockSpec((1, 36), lambda i: (0, 0)),
            pl.BlockSpec((36, DW), lambda i: (0, 0)),
            pl.BlockSpec((1, DW), lambda i: (0, 0)),
        ],
        out_specs=pl.BlockSpec((NB, DW), lambda i: (i, 0)),
        out_shape=jax.ShapeDtypeStruct((EP, DW), jnp.float32),
    )(ef, tb, ts, we, be, wm_pad, bm_pad)


def _p_body(x_ref, w_ref, o_ref):
    o_ref[...] = jnp.dot(x_ref[...], w_ref[...], precision=_HI,
                         preferred_element_type=jnp.float32)


def _p_mat(x, w_pad):
    grid = NP // NB
    return pl.pallas_call(
        _p_body,
        grid=(grid,),
        in_specs=[
            pl.BlockSpec((NB, D), lambda i: (i, 0)),
            pl.BlockSpec((D, DW), lambda i: (0, 0)),
        ],
        out_specs=pl.BlockSpec((NB, DW), lambda i: (i, 0)),
        out_shape=jax.ShapeDtypeStruct((NP, DW), jnp.float32),
    )(x, w_pad)


def _upd_body(x_ref, a_ref, w1_ref, w2_ref, b_ref, o_ref):
    v = jnp.dot(x_ref[...], w1_ref[...], precision=_HI,
                preferred_element_type=jnp.float32)
    v += jnp.dot(a_ref[...], w2_ref[...], precision=_HI,
                 preferred_element_type=jnp.float32)
    o_ref[...] = jnp.maximum(v + b_ref[...], 0.0)


def _update(x, agg, wu1, wu2, b):
    grid = NP // NB
    return pl.pallas_call(
        _upd_body,
        grid=(grid,),
        in_specs=[
            pl.BlockSpec((NB, D), lambda i: (i, 0)),
            pl.BlockSpec((NB, D), lambda i: (i, 0)),
            pl.BlockSpec((D, D), lambda i: (0, 0)),
            pl.BlockSpec((D, D), lambda i: (0, 0)),
            pl.BlockSpec((1, D), lambda i: (0, 0)),
        ],
        out_specs=pl.BlockSpec((NB, D), lambda i: (i, 0)),
        out_shape=jax.ShapeDtypeStruct((NP, D), jnp.float32),
    )(x, agg, wu1, wu2, b)


def _sc_round(P, Q, snd, rcv):
    mesh = plsc.VectorSubcoreMesh(core_axis_name="c", subcore_axis_name="s")

    @functools.partial(
        pl.kernel,
        mesh=mesh,
        out_type=jax.ShapeDtypeStruct((NP, D), jnp.float32),
        compiler_params=pltpu.CompilerParams(use_tc_tiling_on_sc=False),
        scratch_types=[
            pltpu.VMEM((EPB,), jnp.int32),
            pltpu.VMEM((EPB,), jnp.int32),
            pltpu.VMEM((EPB,), jnp.int32),
            pltpu.VMEM((EPB,), jnp.int32),
            pltpu.VMEM((EPB,), jnp.int32),
            pltpu.VMEM((EPB, DW), jnp.float32),
            pltpu.VMEM((EPB, DW), jnp.float32),
            pltpu.VMEM((EPB, DW), jnp.float32),
            pltpu.VMEM((EPB, DW), jnp.float32),
            pltpu.VMEM((EPB, D), jnp.float32),
            pltpu.VMEM((CO_CHUNK, D), jnp.float32),
            pltpu.VMEM_SHARED((HALF + 8, D), jnp.float32),
            pltpu.SemaphoreType.DMA,
            pltpu.SemaphoreType.DMA,
            pltpu.SemaphoreType.DMA,
            pltpu.SemaphoreType.DMA,
        ],
    )
    def k(p_h, q_h, s_h, r_h, out_h, sidx0, ridx0, sidx1, ridx1, lidx,
          prow0, qrow0, prow1, qrow1, mrow, cbuf, acc,
          gsem0, qsem0, gsem1, qsem1):
        cid = lax.axis_index("c")
        sid = lax.axis_index("s")
        coff = cid * HALF

        def zrow(i, c):
            for j in MSG_OFFS:
                cbuf[i, pl.ds(j, 16)] = jnp.zeros((16,), jnp.float32)
            return c
        lax.fori_loop(0, CO_CHUNK, zrow, 0)

        def zchunk(kk, c):
            pltpu.sync_copy(cbuf, acc.at[pl.ds(sid * ROWS_PER_SUB
                                               + kk * CO_CHUNK, CO_CHUNK)])
            return c
        lax.fori_loop(0, ROWS_PER_SUB // CO_CHUNK, zchunk, 0)

        @pl.when(sid == 0)
        def _():
            pltpu.sync_copy(cbuf.at[pl.ds(0, 8)], acc.at[pl.ds(HALF, 8)])

        plsc.subcore_barrier()

        sub_base = sid * (2 * PAIRS * EPB)

        def fetch(base, sidx, ridx, prow, qrow, gsem, qsem):
            pltpu.sync_copy(s_h.at[pl.ds(base, EPB)], sidx)
            pltpu.sync_copy(r_h.at[pl.ds(base, EPB)], ridx)
            pltpu.async_copy(p_h.at[sidx], prow, gsem)
            pltpu.async_copy(q_h.at[pl.ds(base, EPB)], qrow, qsem)

        def consume(sidx, ridx, prow, qrow, gsem, qsem):
            pltpu.make_async_copy(p_h.at[sidx], prow, gsem).wait()
            pltpu.make_async_copy(q_h.at[pl.ds(0, EPB)], qrow, qsem).wait()

            def lix(i, c2):
                r = ridx[pl.ds(i * 16, 16)] - coff
                ok = (r >= 0) & (r < HALF)
                lidx[pl.ds(i * 16, 16)] = jnp.where(ok, r, HALF)
                return c2
            lax.fori_loop(0, EPB // 16, lix, 0)

            def rrow(i, c2):
                for j in MSG_OFFS:
                    v = prow[i, pl.ds(j, 16)] + qrow[i, pl.ds(j, 16)]
                    mrow[i, pl.ds(j, 16)] = jnp.maximum(v, 0.0)
                return c2
            lax.fori_loop(0, EPB, rrow, 0)

            pltpu.sync_copy(mrow, acc.at[lidx], add=True)

        fetch(sub_base, sidx0, ridx0, prow0, qrow0, gsem0, qsem0)

        def body(t, c):
            fetch(sub_base + (2 * t + 1) * EPB,
                  sidx1, ridx1, prow1, qrow1, gsem1, qsem1)
            consume(sidx0, ridx0, prow0, qrow0, gsem0, qsem0)

            @pl.when(t < PAIRS - 1)
            def _():
                fetch(sub_base + (2 * t + 2) * EPB,
                      sidx0, ridx0, prow0, qrow0, gsem0, qsem0)

            consume(sidx1, ridx1, prow1, qrow1, gsem1, qsem1)
            return c
        lax.fori_loop(0, PAIRS, body, 0)

        plsc.subcore_barrier()

        def cout(kk, c):
            off = sid * ROWS_PER_SUB + kk * CO_CHUNK
            pltpu.sync_copy(acc.at[pl.ds(off, CO_CHUNK)], cbuf)
            pltpu.sync_copy(cbuf, out_h.at[pl.ds(coff + off, CO_CHUNK)])
            return c
        lax.fori_loop(0, ROWS_PER_SUB // CO_CHUNK, cout, 0)

    return k(P, Q, snd, rcv)


def _pool1_body(x_ref, wp_ref, idx_ref, lg_ref, mx_ref):
    @pl.when(pl.program_id(0) == 0)
    def _():
        mx_ref[...] = jnp.full((1, G), -1e30, jnp.float32)
    x = x_ref[...]
    idx = idx_ref[...].reshape(NB, 1)
    lg = jnp.dot(x, wp_ref[...], precision=_HI,
                 preferred_element_type=jnp.float32)
    lgm = jnp.where(idx < G, lg, 0.0)
    lg_ref[...] = lgm
    oh = _onehot(idx, G)
    cand = jnp.where(oh > 0.0, lgm, -1e30)
    mx_ref[...] = jnp.maximum(mx_ref[...], jnp.max(cand, axis=0)[None, :])


def _pool1(x, wp, idx3):
    grid = NP // NB
    return pl.pallas_call(
        _pool1_body,
        grid=(grid,),
        in_specs=[
            pl.BlockSpec((NB, D), lambda i: (i, 0)),
            pl.BlockSpec((D, 1), lambda i: (0, 0)),
            pl.BlockSpec((1, 1, NB), lambda i: (i, 0, 0)),
        ],
        out_specs=[
            pl.BlockSpec((NB, 1), lambda i: (i, 0)),
            pl.BlockSpec((1, G), lambda i: (0, 0)),
        ],
        out_shape=[
            jax.ShapeDtypeStruct((NP, 1), jnp.float32),
            jax.ShapeDtypeStruct((1, G), jnp.float32),
        ],
    )(x, wp, idx3)


def _pool2_body(x_ref, lg_ref, mx_ref, idx_ref, s_ref, dn_ref):
    @pl.when(pl.program_id(0) == 0)
    def _():
        s_ref[...] = jnp.zeros((G, D), jnp.float32)
        dn_ref[...] = jnp.zeros((G, 1), jnp.float32)
    x = x_ref[...]
    idx = idx_ref[...].reshape(NB, 1)
    oh = _onehot(idx, G)
    mxn = jnp.dot(oh, mx_ref[...].reshape(G, 1), precision=_HI,
                  preferred_element_type=jnp.float32)
    ex = jnp.exp(lg_ref[...] - mxn)
    s_ref[...] += jnp.dot(oh.T, ex * x, precision=_HI,
                          preferred_element_type=jnp.float32)
    dn_ref[...] += jnp.dot(oh.T, ex, precision=_HI,
                           preferred_element_type=jnp.float32)


def _pool2(x, lg, mx, idx3):
    grid = NP // NB
    return pl.pallas_call(
        _pool2_body,
        grid=(grid,),
        in_specs=[
            pl.BlockSpec((NB, D), lambda i: (i, 0)),
            pl.BlockSpec((NB, 1), lambda i: (i, 0)),
            pl.BlockSpec((1, G), lambda i: (0, 0)),
            pl.BlockSpec((1, 1, NB), lambda i: (i, 0, 0)),
        ],
        out_specs=[
            pl.BlockSpec((G, D), lambda i: (0, 0)),
            pl.BlockSpec((G, 1), lambda i: (0, 0)),
        ],
        out_shape=[
            jax.ShapeDtypeStruct((G, D), jnp.float32),
            jax.ShapeDtypeStruct((G, 1), jnp.float32),
        ],
    )(x, lg, mx, idx3)


def _mlp_body(s_ref, dn_ref, w1_ref, b1_ref, w2_ref, b2_ref, w3_ref, b3_ref,
              w4_ref, b4_ref, lg_ref, lb_ref, wo_ref, bo_ref, o_ref):
    g = s_ref[...] / (dn_ref[...] + 1e-9)
    ge = g.reshape(G // 2, 2, D)[:, 0, :]
    x = jnp.maximum(jnp.dot(ge, w1_ref[...], precision=_HI,
                            preferred_element_type=jnp.float32)
                    + b1_ref[...], 0.0)
    x = jnp.maximum(jnp.dot(x, w2_ref[...], precision=_HI,
                            preferred_element_type=jnp.float32)
                    + b2_ref[...], 0.0)
    x = jnp.maximum(jnp.dot(x, w3_ref[...], precision=_HI,
                            preferred_element_type=jnp.float32)
                    + b3_ref[...], 0.0)
    x = jnp.dot(x, w4_ref[...], precision=_HI,
                preferred_element_type=jnp.float32) + b4_ref[...]
    m = jnp.mean(x, axis=-1, keepdims=True)
    v = jnp.mean((x - m) * (x - m), axis=-1, keepdims=True)
    xe = lg_ref[...] * (x - m) * jax.lax.rsqrt(v + 1e-6) + lb_ref[...]
    o_ref[...] = jnp.dot(xe, wo_ref[...], precision=_HI,
                         preferred_element_type=jnp.float32) + bo_ref[...]


def _mlp(s, dn, w1, b1, w2, b2, w3, b3, w4, b4, lng, lnb, wo, bo):
    H = 392
    NCLS = 12
    return pl.pallas_call(
        _mlp_body,
        in_specs=[
            pl.BlockSpec((G, D), lambda: (0, 0)),
            pl.BlockSpec((G, 1), lambda: (0, 0)),
            pl.BlockSpec((D, H), lambda: (0, 0)),
            pl.BlockSpec((1, H), lambda: (0, 0)),
            pl.BlockSpec((H, H), lambda: (0, 0)),
            pl.BlockSpec((1, H), lambda: (0, 0)),
            pl.BlockSpec((H, H), lambda: (0, 0)),
            pl.BlockSpec((1, H), lambda: (0, 0)),
            pl.BlockSpec((H, H), lambda: (0, 0)),
            pl.BlockSpec((1, H), lambda: (0, 0)),
            pl.BlockSpec((1, H), lambda: (0, 0)),
            pl.BlockSpec((1, H), lambda: (0, 0)),
            pl.BlockSpec((H, NCLS), lambda: (0, 0)),
            pl.BlockSpec((1, NCLS), lambda: (0, 0)),
        ],
        out_specs=pl.BlockSpec((G // 2, NCLS), lambda: (0, 0)),
        out_shape=jax.ShapeDtypeStruct((G // 2, NCLS), jnp.float32),
    )(s, dn, w1, b1, w2, b2, w3, b3, w4, b4, lng, lnb, wo, bo)


def kernel(node_feats, edge_feats, senders, receivers, node_graph_idx,
           atomic_num_table, chiral_table, hybrid_table, bond_type_table,
           stereo_table, W_xproj, b_xproj, W_eproj, b_eproj, W_msg, b_msg,
           W_upd, b_upd, W_pool, W1, b1, W2, b2, W3, b3, W4, b4, ln_g, ln_b,
           W_out, b_out, deterministic, return_embeddings):
    f32 = jnp.float32
    nf = jnp.zeros((NP, 6), f32).at[:NN].set(node_feats.astype(f32))
    ef = jnp.zeros((EP, 4), f32).at[:EG].set(edge_feats.astype(f32))
    snd = jnp.zeros((EP,), jnp.int32).at[:EG].set(senders.astype(jnp.int32))
    rcv = jnp.full((EP,), NP, jnp.int32).at[:EG].set(
        receivers.astype(jnp.int32))
    gidx = jnp.full((NP,), G, jnp.int32).at[:NN].set(
        node_graph_idx.astype(jnp.int32))
    gidx3 = gidx.reshape(NP // NB, 1, NB)

    wm_x = jnp.zeros((D, DW), f32).at[:, :D].set(W_msg[:D].astype(f32))
    wm_e = jnp.zeros((36, DW), f32).at[:, :D].set(W_msg[D:].astype(f32))
    bm = jnp.zeros((1, DW), f32).at[0, :D].set(b_msg.astype(f32))
    wu1 = W_upd[:D].astype(f32)
    wu2 = W_upd[D:].astype(f32)

    x = _node_embed(nf, atomic_num_table.astype(f32),
                    chiral_table.astype(f32), hybrid_table.astype(f32),
                    W_xproj.astype(f32), b_xproj.astype(f32).reshape(1, D))
    q = _edge_q(ef, bond_type_table.astype(f32), stereo_table.astype(f32),
                W_eproj.astype(f32), b_eproj.astype(f32).reshape(1, 36),
                wm_e, bm)

    bu = b_upd.astype(f32).reshape(1, D)
    for _ in range(5):
        p = _p_mat(x, wm_x)
        agg = _sc_round(p, q, snd, rcv)
        x = _update(x, agg, wu1, wu2, bu)

    lg, mx = _pool1(x, W_pool.astype(f32), gidx3)
    s, dn = _pool2(x, lg, mx, gidx3)
    out = _mlp(s, dn, W1.astype(f32), b1.astype(f32).reshape(1, 392),
               W2.astype(f32), b2.astype(f32).reshape(1, 392),
               W3.astype(f32), b3.astype(f32).reshape(1, 392),
               W4.astype(f32), b4.astype(f32).reshape(1, 392),
               ln_g.astype(f32).reshape(1, 392), ln_b.astype(f32).reshape(1, 392),
               W_out.astype(f32), b_out.astype(f32).reshape(1, 12))
    return out + jnp.zeros((), out.dtype) * return_embeddings


# speedup vs baseline: 1.8129x; 1.4289x over previous
import functools
import jax
import jax.numpy as jnp
from jax import lax
from jax.experimental import pallas as pl
from jax.experimental.pallas import tpu as pltpu
from jax.experimental.pallas import tpu_sc as plsc

NN = 50000
EG = 800000
G = 1024
D = 72
DW = 80
NB = 512
NP = 50176
EP = 800768
HALF = NP // 2
EPB = 32
SUBS = 16
PAIRS = EP // (SUBS * EPB * 2)
CO_CHUNK = 49
ROWS_PER_SUB = HALF // SUBS
MSG_OFFS = (0, 16, 32, 48, 56)

_HI = jax.lax.Precision.HIGHEST


def _onehot(ids, width):
    iot = lax.broadcasted_iota(jnp.int32, (ids.shape[0], width), 1)
    return (ids == iot).astype(jnp.float32)


def _node_embed_body(nf_ref, ta_ref, tc_ref, th_ref, wx_ref, bx_ref, o_ref):
    nf = nf_ref[...]
    ai = nf[:, 0:1].astype(jnp.int32)
    ci = nf[:, 1:2].astype(jnp.int32)
    hi = nf[:, 2:3].astype(jnp.int32)
    x = jnp.dot(_onehot(ai, 32), ta_ref[...], precision=_HI,
                preferred_element_type=jnp.float32)
    x += jnp.dot(_onehot(ci, 32), tc_ref[...], precision=_HI,
                 preferred_element_type=jnp.float32)
    x += jnp.dot(_onehot(hi, 32), th_ref[...], precision=_HI,
                 preferred_element_type=jnp.float32)
    x += jnp.dot(nf[:, 3:6], wx_ref[...], precision=_HI,
                 preferred_element_type=jnp.float32)
    o_ref[...] = x + bx_ref[...]


def _node_embed(nf, ta, tc, th, wx, bx):
    grid = NP // NB
    return pl.pallas_call(
        _node_embed_body,
        grid=(grid,),
        in_specs=[
            pl.BlockSpec((NB, 6), lambda i: (i, 0)),
            pl.BlockSpec((32, D), lambda i: (0, 0)),
            pl.BlockSpec((32, D), lambda i: (0, 0)),
            pl.BlockSpec((32, D), lambda i: (0, 0)),
            pl.BlockSpec((3, D), lambda i: (0, 0)),
            pl.BlockSpec((1, D), lambda i: (0, 0)),
        ],
        out_specs=pl.BlockSpec((NB, D), lambda i: (i, 0)),
        out_shape=jax.ShapeDtypeStruct((NP, D), jnp.float32),
    )(nf, ta, tc, th, wx, bx)


def _edge_q_body(ef_ref, tb_ref, ts_ref, we_ref, be_ref, wm_ref, bm_ref, o_ref):
    ef = ef_ref[...]
    bi = ef[:, 0:1].astype(jnp.int32)
    si = ef[:, 1:2].astype(jnp.int32)
    e = jnp.dot(_onehot(bi, 32), tb_ref[...], precision=_HI,
                preferred_element_type=jnp.float32)
    e += jnp.dot(_onehot(si, 32), ts_ref[...], precision=_HI,
                 preferred_element_type=jnp.float32)
    e += jnp.dot(ef[:, 2:4], we_ref[...], precision=_HI,
                 preferred_element_type=jnp.float32)
    e += be_ref[...]
    o_ref[...] = jnp.dot(e, wm_ref[...], precision=_HI,
                         preferred_element_type=jnp.float32) + bm_ref[...]


def _edge_q(ef, tb, ts, we, be, wm_pad, bm_pad):
    grid = EP // NB
    return pl.pallas_call(
        _edge_q_body,
        grid=(grid,),
        in_specs=[
            pl.BlockSpec((NB, 4), lambda i: (i, 0)),
            pl.BlockSpec((32, 36), lambda i: (0, 0)),
            pl.BlockSpec((32, 36), lambda i: (0, 0)),
            pl.BlockSpec((2, 36), lambda i: (0, 0)),
            pl.BlockSpec((1, 36), lambda i: (0, 0)),
            pl.BlockSpec((36, DW), lambda i: (0, 0)),
            pl.BlockSpec((1, DW), lambda i: (0, 0)),
        ],
        out_specs=pl.BlockSpec((NB, DW), lambda i: (i, 0)),
        out_shape=jax.ShapeDtypeStruct((EP, DW), jnp.float32),
    )(ef, tb, ts, we, be, wm_pad, bm_pad)


def _p_body(x_ref, w_ref, o_ref):
    o_ref[...] = jnp.dot(x_ref[...], w_ref[...], precision=_HI,
                         preferred_element_type=jnp.float32)


def _p_mat(x, w_pad):
    grid = NP // NB
    return pl.pallas_call(
        _p_body,
        grid=(grid,),
        in_specs=[
            pl.BlockSpec((NB, D), lambda i: (i, 0)),
            pl.BlockSpec((D, DW), lambda i: (0, 0)),
        ],
        out_specs=pl.BlockSpec((NB, DW), lambda i: (i, 0)),
        out_shape=jax.ShapeDtypeStruct((NP, DW), jnp.float32),
    )(x, w_pad)


def _upd_body(x_ref, a_ref, w1_ref, w2_ref, b_ref, o_ref):
    v = jnp.dot(x_ref[...], w1_ref[...], precision=_HI,
                preferred_element_type=jnp.float32)
    v += jnp.dot(a_ref[...], w2_ref[...], precision=_HI,
                 preferred_element_type=jnp.float32)
    o_ref[...] = jnp.maximum(v + b_ref[...], 0.0)


def _update(x, agg, wu1, wu2, b):
    grid = NP // NB
    return pl.pallas_call(
        _upd_body,
        grid=(grid,),
        in_specs=[
            pl.BlockSpec((NB, D), lambda i: (i, 0)),
            pl.BlockSpec((NB, D), lambda i: (i, 0)),
            pl.BlockSpec((D, D), lambda i: (0, 0)),
            pl.BlockSpec((D, D), lambda i: (0, 0)),
            pl.BlockSpec((1, D), lambda i: (0, 0)),
        ],
        out_specs=pl.BlockSpec((NB, D), lambda i: (i, 0)),
        out_shape=jax.ShapeDtypeStruct((NP, D), jnp.float32),
    )(x, agg, wu1, wu2, b)


def _sc_round(P, Q, snd, rcv):
    mesh = plsc.VectorSubcoreMesh(core_axis_name="c", subcore_axis_name="s")

    @functools.partial(
        pl.kernel,
        mesh=mesh,
        out_type=jax.ShapeDtypeStruct((NP, D), jnp.float32),
        compiler_params=pltpu.CompilerParams(use_tc_tiling_on_sc=False),
        scratch_types=[
            pltpu.VMEM((EPB,), jnp.int32),
            pltpu.VMEM((EPB,), jnp.int32),
            pltpu.VMEM((EPB,), jnp.int32),
            pltpu.VMEM((EPB,), jnp.int32),
            pltpu.VMEM((EPB,), jnp.int32),
            pltpu.VMEM((EPB, DW), jnp.float32),
            pltpu.VMEM((EPB, DW), jnp.float32),
            pltpu.VMEM((EPB, DW), jnp.float32),
            pltpu.VMEM((EPB, DW), jnp.float32),
            pltpu.VMEM((EPB, D), jnp.float32),
            pltpu.VMEM((CO_CHUNK, D), jnp.float32),
            pltpu.VMEM_SHARED((HALF + 8, D), jnp.float32),
            pltpu.SemaphoreType.DMA,
            pltpu.SemaphoreType.DMA,
            pltpu.SemaphoreType.DMA,
            pltpu.SemaphoreType.DMA,
            pltpu.SemaphoreType.DMA,
            pltpu.SemaphoreType.DMA,
        ],
    )
    def k(p_h, q_h, s_h, r_h, out_h, sidx0, ridx0, sidx1, ridx1, lidx,
          prow0, qrow0, prow1, qrow1, mrow, cbuf, acc,
          gsem0, qsem0, gsem1, qsem1, isem0, isem1):
        cid = lax.axis_index("c")
        sid = lax.axis_index("s")
        coff = cid * HALF

        def zrow(i, c):
            for j in MSG_OFFS:
                cbuf[i, pl.ds(j, 16)] = jnp.zeros((16,), jnp.float32)
            return c
        lax.fori_loop(0, CO_CHUNK, zrow, 0)

        def zchunk(kk, c):
            pltpu.sync_copy(cbuf, acc.at[pl.ds(sid * ROWS_PER_SUB
                                               + kk * CO_CHUNK, CO_CHUNK)])
            return c
        lax.fori_loop(0, ROWS_PER_SUB // CO_CHUNK, zchunk, 0)

        @pl.when(sid == 0)
        def _():
            pltpu.sync_copy(cbuf.at[pl.ds(0, 8)], acc.at[pl.ds(HALF, 8)])

        plsc.subcore_barrier()

        sub_base = sid * (2 * PAIRS * EPB)
        last0 = sub_base + (2 * PAIRS - 2) * EPB
        last1 = sub_base + (2 * PAIRS - 1) * EPB

        def prefetch_idx(base, sidx, ridx, isem):
            pltpu.async_copy(s_h.at[pl.ds(base, EPB)], sidx, isem)
            pltpu.async_copy(r_h.at[pl.ds(base, EPB)], ridx, isem)

        def fetch(base, sidx, ridx, prow, qrow, gsem, qsem, isem):
            pltpu.make_async_copy(s_h.at[pl.ds(0, EPB)], sidx, isem).wait()
            pltpu.make_async_copy(r_h.at[pl.ds(0, EPB)], ridx, isem).wait()
            pltpu.async_copy(p_h.at[sidx], prow, gsem)
            pltpu.async_copy(q_h.at[pl.ds(base, EPB)], qrow, qsem)

        def consume(base, last, sidx, ridx, prow, qrow, gsem, qsem, isem):
            pltpu.make_async_copy(p_h.at[sidx], prow, gsem).wait()
            pltpu.make_async_copy(q_h.at[pl.ds(0, EPB)], qrow, qsem).wait()

            def lix(i, c2):
                r = ridx[pl.ds(i * 16, 16)] - coff
                ok = (r >= 0) & (r < HALF)
                lidx[pl.ds(i * 16, 16)] = jnp.where(ok, r, HALF)
                return c2
            lax.fori_loop(0, EPB // 16, lix, 0)

            @pl.when(base < last)
            def _():
                prefetch_idx(base + 2 * EPB, sidx, ridx, isem)

            def rrow(i, c2):
                for jj in range(4):
                    for j in MSG_OFFS:
                        v = (prow[i * 4 + jj, pl.ds(j, 16)]
                             + qrow[i * 4 + jj, pl.ds(j, 16)])
                        mrow[i * 4 + jj, pl.ds(j, 16)] = jnp.maximum(v, 0.0)
                return c2
            lax.fori_loop(0, EPB // 4, rrow, 0)

            pltpu.sync_copy(mrow, acc.at[lidx], add=True)

        prefetch_idx(sub_base, sidx0, ridx0, isem0)
        prefetch_idx(sub_base + EPB, sidx1, ridx1, isem1)
        fetch(sub_base, sidx0, ridx0, prow0, qrow0, gsem0, qsem0, isem0)

        def body(t, c):
            b0 = sub_base + (2 * t) * EPB
            b1 = sub_base + (2 * t + 1) * EPB
            fetch(b1, sidx1, ridx1, prow1, qrow1, gsem1, qsem1, isem1)
            consume(b0, last0, sidx0, ridx0, prow0, qrow0, gsem0, qsem0,
                    isem0)

            @pl.when(t < PAIRS - 1)
            def _():
                fetch(b0 + 2 * EPB, sidx0, ridx0, prow0, qrow0, gsem0,
                      qsem0, isem0)

            consume(b1, last1, sidx1, ridx1, prow1, qrow1, gsem1, qsem1,
                    isem1)
            return c
        lax.fori_loop(0, PAIRS, body, 0)

        plsc.subcore_barrier()

        def cout(kk, c):
            off = sid * ROWS_PER_SUB + kk * CO_CHUNK
            pltpu.sync_copy(acc.at[pl.ds(off, CO_CHUNK)], cbuf)
            pltpu.sync_copy(cbuf, out_h.at[pl.ds(coff + off, CO_CHUNK)])
            return c
        lax.fori_loop(0, ROWS_PER_SUB // CO_CHUNK, cout, 0)

    return k(P, Q, snd, rcv)


def _pool1_body(x_ref, wp_ref, idx_ref, lg_ref, mx_ref):
    @pl.when(pl.program_id(0) == 0)
    def _():
        mx_ref[...] = jnp.full((1, G), -1e30, jnp.float32)
    x = x_ref[...]
    idx = idx_ref[...].reshape(NB, 1)
    lg = jnp.dot(x, wp_ref[...], precision=_HI,
                 preferred_element_type=jnp.float32)
    lgm = jnp.where(idx < G, lg, 0.0)
    lg_ref[...] = lgm
    oh = _onehot(idx, G)
    cand = jnp.where(oh > 0.0, lgm, -1e30)
    mx_ref[...] = jnp.maximum(mx_ref[...], jnp.max(cand, axis=0)[None, :])


def _pool1(x, wp, idx3):
    grid = NP // NB
    return pl.pallas_call(
        _pool1_body,
        grid=(grid,),
        in_specs=[
            pl.BlockSpec((NB, D), lambda i: (i, 0)),
            pl.BlockSpec((D, 1), lambda i: (0, 0)),
            pl.BlockSpec((1, 1, NB), lambda i: (i, 0, 0)),
        ],
        out_specs=[
            pl.BlockSpec((NB, 1), lambda i: (i, 0)),
            pl.BlockSpec((1, G), lambda i: (0, 0)),
        ],
        out_shape=[
            jax.ShapeDtypeStruct((NP, 1), jnp.float32),
            jax.ShapeDtypeStruct((1, G), jnp.float32),
        ],
    )(x, wp, idx3)


def _pool2_body(x_ref, lg_ref, mx_ref, idx_ref, s_ref, dn_ref):
    @pl.when(pl.program_id(0) == 0)
    def _():
        s_ref[...] = jnp.zeros((G, D), jnp.float32)
        dn_ref[...] = jnp.zeros((G, 1), jnp.float32)
    x = x_ref[...]
    idx = idx_ref[...].reshape(NB, 1)
    oh = _onehot(idx, G)
    mxn = jnp.dot(oh, mx_ref[...].reshape(G, 1), precision=_HI,
                  preferred_element_type=jnp.float32)
    ex = jnp.exp(lg_ref[...] - mxn)
    s_ref[...] += jnp.dot(oh.T, ex * x, precision=_HI,
                          preferred_element_type=jnp.float32)
    dn_ref[...] += jnp.dot(oh.T, ex, precision=_HI,
                           preferred_element_type=jnp.float32)


def _pool2(x, lg, mx, idx3):
    grid = NP // NB
    return pl.pallas_call(
        _pool2_body,
        grid=(grid,),
        in_specs=[
            pl.BlockSpec((NB, D), lambda i: (i, 0)),
            pl.BlockSpec((NB, 1), lambda i: (i, 0)),
            pl.BlockSpec((1, G), lambda i: (0, 0)),
            pl.BlockSpec((1, 1, NB), lambda i: (i, 0, 0)),
        ],
        out_specs=[
            pl.BlockSpec((G, D), lambda i: (0, 0)),
            pl.BlockSpec((G, 1), lambda i: (0, 0)),
        ],
        out_shape=[
            jax.ShapeDtypeStruct((G, D), jnp.float32),
            jax.ShapeDtypeStruct((G, 1), jnp.float32),
        ],
    )(x, lg, mx, idx3)


def _mlp_body(s_ref, dn_ref, w1_ref, b1_ref, w2_ref, b2_ref, w3_ref, b3_ref,
              w4_ref, b4_ref, lg_ref, lb_ref, wo_ref, bo_ref, o_ref):
    g = s_ref[...] / (dn_ref[...] + 1e-9)
    ge = g.reshape(G // 2, 2, D)[:, 0, :]
    x = jnp.maximum(jnp.dot(ge, w1_ref[...], precision=_HI,
                            preferred_element_type=jnp.float32)
                    + b1_ref[...], 0.0)
    x = jnp.maximum(jnp.dot(x, w2_ref[...], precision=_HI,
                            preferred_element_type=jnp.float32)
                    + b2_ref[...], 0.0)
    x = jnp.maximum(jnp.dot(x, w3_ref[...], precision=_HI,
                            preferred_element_type=jnp.float32)
                    + b3_ref[...], 0.0)
    x = jnp.dot(x, w4_ref[...], precision=_HI,
                preferred_element_type=jnp.float32) + b4_ref[...]
    m = jnp.mean(x, axis=-1, keepdims=True)
    v = jnp.mean((x - m) * (x - m), axis=-1, keepdims=True)
    xe = lg_ref[...] * (x - m) * jax.lax.rsqrt(v + 1e-6) + lb_ref[...]
    o_ref[...] = jnp.dot(xe, wo_ref[...], precision=_HI,
                         preferred_element_type=jnp.float32) + bo_ref[...]


def _mlp(s, dn, w1, b1, w2, b2, w3, b3, w4, b4, lng, lnb, wo, bo):
    H = 392
    NCLS = 12
    return pl.pallas_call(
        _mlp_body,
        in_specs=[
            pl.BlockSpec((G, D), lambda: (0, 0)),
            pl.BlockSpec((G, 1), lambda: (0, 0)),
            pl.BlockSpec((D, H), lambda: (0, 0)),
            pl.BlockSpec((1, H), lambda: (0, 0)),
            pl.BlockSpec((H, H), lambda: (0, 0)),
            pl.BlockSpec((1, H), lambda: (0, 0)),
            pl.BlockSpec((H, H), lambda: (0, 0)),
            pl.BlockSpec((1, H), lambda: (0, 0)),
            pl.BlockSpec((H, H), lambda: (0, 0)),
            pl.BlockSpec((1, H), lambda: (0, 0)),
            pl.BlockSpec((1, H), lambda: (0, 0)),
            pl.BlockSpec((1, H), lambda: (0, 0)),
            pl.BlockSpec((H, NCLS), lambda: (0, 0)),
            pl.BlockSpec((1, NCLS), lambda: (0, 0)),
        ],
        out_specs=pl.BlockSpec((G // 2, NCLS), lambda: (0, 0)),
        out_shape=jax.ShapeDtypeStruct((G // 2, NCLS), jnp.float32),
    )(s, dn, w1, b1, w2, b2, w3, b3, w4, b4, lng, lnb, wo, bo)


def kernel(node_feats, edge_feats, senders, receivers, node_graph_idx,
           atomic_num_table, chiral_table, hybrid_table, bond_type_table,
           stereo_table, W_xproj, b_xproj, W_eproj, b_eproj, W_msg, b_msg,
           W_upd, b_upd, W_pool, W1, b1, W2, b2, W3, b3, W4, b4, ln_g, ln_b,
           W_out, b_out, deterministic, return_embeddings):
    f32 = jnp.float32
    nf = jnp.zeros((NP, 6), f32).at[:NN].set(node_feats.astype(f32))
    ef = jnp.zeros((EP, 4), f32).at[:EG].set(edge_feats.astype(f32))
    snd = jnp.zeros((EP,), jnp.int32).at[:EG].set(senders.astype(jnp.int32))
    rcv = jnp.full((EP,), NP, jnp.int32).at[:EG].set(
        receivers.astype(jnp.int32))
    gidx = jnp.full((NP,), G, jnp.int32).at[:NN].set(
        node_graph_idx.astype(jnp.int32))
    gidx3 = gidx.reshape(NP // NB, 1, NB)

    wm_x = jnp.zeros((D, DW), f32).at[:, :D].set(W_msg[:D].astype(f32))
    wm_e = jnp.zeros((36, DW), f32).at[:, :D].set(W_msg[D:].astype(f32))
    bm = jnp.zeros((1, DW), f32).at[0, :D].set(b_msg.astype(f32))
    wu1 = W_upd[:D].astype(f32)
    wu2 = W_upd[D:].astype(f32)

    x = _node_embed(nf, atomic_num_table.astype(f32),
                    chiral_table.astype(f32), hybrid_table.astype(f32),
                    W_xproj.astype(f32), b_xproj.astype(f32).reshape(1, D))
    q = _edge_q(ef, bond_type_table.astype(f32), stereo_table.astype(f32),
                W_eproj.astype(f32), b_eproj.astype(f32).reshape(1, 36),
                wm_e, bm)

    bu = b_upd.astype(f32).reshape(1, D)
    for _ in range(5):
        p = _p_mat(x, wm_x)
        agg = _sc_round(p, q, snd, rcv)
        x = _update(x, agg, wu1, wu2, bu)

    lg, mx = _pool1(x, W_pool.astype(f32), gidx3)
    s, dn = _pool2(x, lg, mx, gidx3)
    out = _mlp(s, dn, W1.astype(f32), b1.astype(f32).reshape(1, 392),
               W2.astype(f32), b2.astype(f32).reshape(1, 392),
               W3.astype(f32), b3.astype(f32).reshape(1, 392),
               W4.astype(f32), b4.astype(f32).reshape(1, 392),
               ln_g.astype(f32).reshape(1, 392), ln_b.astype(f32).reshape(1, 392),
               W_out.astype(f32), b_out.astype(f32).reshape(1, 12))
    return out + jnp.zeros((), out.dtype) * return_embeddings
